# baseline (speedup 1.0000x reference)
import functools

import jax
import jax.numpy as jnp
from jax import lax
from jax.experimental import pallas as pl
from jax.experimental.pallas import tpu as pltpu
from jax.experimental.pallas import tpu_sc as plsc

_N = 10000
_E = 320000
_D = 128
_H = 128

_NC = 2
_NS = 16
_L = 16
_NW = _NC * _NS
_EPW = _E // _NW
_C = 80
_NCHUNK = _EPW // _C
_NP = 10240
_U = 80
_RPT = _NP // _NS


def _sc_agg_body(h_hbm, src_hbm, dst_hbm, agg_out,
                 src_v0, dst_v0, sdst_v0, rows_v0,
                 src_v1, dst_v1, sdst_v1, rows_v1,
                 agg_sh, sem_i0, sem_i1, sem_g0, sem_g1, sem_s0, sem_s1):
    zbuf = rows_v0

    cid = lax.axis_index("c")
    sid = lax.axis_index("s")
    wid = sid * _NC + cid

    zeros16 = jnp.zeros((_L,), jnp.float32)

    def _zb(i, c):
        r = i // (_D // _L)
        k = i % (_D // _L)
        zbuf[r, pl.ds(k * _L, _L)] = zeros16
        return c

    lax.fori_loop(0, _U * (_D // _L), _zb, 0)

    r0 = sid * _RPT
    for j in range(_RPT // _U):
        pltpu.sync_copy(zbuf, agg_sh.at[pl.ds(r0 + j * _U, _U), :])

    plsc.subcore_barrier()

    ebase = wid * _EPW
    emax = _E - _C

    bufs = ((src_v0, dst_v0, sdst_v0, rows_v0, sem_i0, sem_g0, sem_s0),
            (src_v1, dst_v1, sdst_v1, rows_v1, sem_i1, sem_g1, sem_s1))

    def _idx_start(j, b):
        off = jnp.minimum(ebase + j * _C, emax)
        ci0 = pltpu.async_copy(src_hbm.at[pl.ds(off, _C)], b[0], b[4])
        ci1 = pltpu.async_copy(dst_hbm.at[pl.ds(off, _C)], b[1], b[4])
        return ci0, ci1

    def _idx_wait(cis):
        cis[0].wait()
        cis[1].wait()

    def _gather_start(b):
        return pltpu.async_copy(h_hbm.at[b[0]], b[3], b[5])

    def _scatter_start(b):
        for k in range(_C // _L):
            b[2][pl.ds(k * _L, _L)] = b[1][pl.ds(k * _L, _L)]
        return pltpu.async_copy(b[3], agg_sh.at[b[2]], b[6], add=True)

    _idx_wait(_idx_start(0, bufs[0]))
    g0 = _gather_start(bufs[0])
    i1 = _idx_start(1, bufs[1])
    _idx_wait(i1)
    g1 = _gather_start(bufs[1])
    g0.wait()
    s0 = _scatter_start(bufs[0])
    i0 = _idx_start(2, bufs[0])
    g1.wait()
    s1 = _scatter_start(bufs[1])
    i1 = _idx_start(3, bufs[1])

    def _pair(i, c):
        a = 2 * i
        _idx_wait(i0)
        s0.wait()
        ga = _gather_start(bufs[0])
        _idx_wait(i1)
        s1.wait()
        gb = _gather_start(bufs[1])
        ga.wait()
        _scatter_start(bufs[0])
        _idx_start(a + 2, bufs[0])
        gb.wait()
        _scatter_start(bufs[1])
        _idx_start(a + 3, bufs[1])
        return c

    lax.fori_loop(1, (_NCHUNK - 1) // 2, _pair, 0)

    _idx_wait(i0)
    s0.wait()
    g0 = _gather_start(bufs[0])
    g0.wait()
    s0 = _scatter_start(bufs[0])
    _idx_wait(i1)
    s1.wait()
    s0.wait()

    plsc.subcore_barrier()

    pltpu.sync_copy(agg_sh.at[pl.ds(r0, _RPT), :],
                    agg_out.at[cid, pl.ds(r0, _RPT), :])


def _sc_deg_body(dst_hbm, deg_out, dst_v, ones_v, zbuf, deg_sh, sem):
    cid = lax.axis_index("c")
    sid = lax.axis_index("s")
    wid = sid * _NC + cid

    zeros16 = jnp.zeros((_L,), jnp.float32)
    ones16 = jnp.ones((_L,), jnp.float32)

    def _fb(i, c):
        r = i // (_D // _L)
        k = i % (_D // _L)
        zbuf[r, pl.ds(k * _L, _L)] = zeros16
        ones_v[r, pl.ds(k * _L, _L)] = ones16
        return c

    lax.fori_loop(0, _U * (_D // _L), _fb, 0)

    r0 = sid * _RPT
    for j in range(_RPT // _U):
        pltpu.sync_copy(zbuf, deg_sh.at[pl.ds(r0 + j * _U, _U), :])

    plsc.subcore_barrier()

    ebase = wid * _EPW

    def _chunk(j, c):
        off = ebase + j * _C
        pltpu.sync_copy(dst_hbm.at[pl.ds(off, _C)], dst_v)
        pltpu.sync_copy(ones_v, deg_sh.at[dst_v], add=True)
        return c

    lax.fori_loop(0, _NCHUNK, _chunk, 0)

    plsc.subcore_barrier()

    pltpu.sync_copy(deg_sh.at[pl.ds(r0, _RPT), :],
                    deg_out.at[cid, pl.ds(r0, _RPT), :])


@functools.lru_cache(maxsize=None)
def _make_sc_deg():
    mesh = plsc.VectorSubcoreMesh(core_axis_name="c", subcore_axis_name="s",
                                  num_cores=_NC, num_subcores=_NS)
    return pl.kernel(
        _sc_deg_body,
        out_type=jax.ShapeDtypeStruct((_NC, _NP, _D), jnp.float32),
        mesh=mesh,
        scratch_types=[
            pltpu.VMEM((_C,), jnp.int32),
            pltpu.VMEM((_C, _D), jnp.float32),
            pltpu.VMEM((_U, _D), jnp.float32),
            pltpu.VMEM_SHARED((_NP, _D), jnp.float32),
            pltpu.SemaphoreType.DMA,
        ],
    )


@functools.lru_cache(maxsize=None)
def _make_sc_agg():
    mesh = plsc.VectorSubcoreMesh(core_axis_name="c", subcore_axis_name="s",
                                  num_cores=_NC, num_subcores=_NS)
    return pl.kernel(
        _sc_agg_body,
        out_type=jax.ShapeDtypeStruct((_NC, _NP, _D), jnp.float32),
        mesh=mesh,
        scratch_types=[
            pltpu.VMEM((_C,), jnp.int32),
            pltpu.VMEM((_C,), jnp.int32),
            pltpu.VMEM((_C,), jnp.int32),
            pltpu.VMEM((_C, _D), jnp.float32),
            pltpu.VMEM((_C,), jnp.int32),
            pltpu.VMEM((_C,), jnp.int32),
            pltpu.VMEM((_C,), jnp.int32),
            pltpu.VMEM((_C, _D), jnp.float32),
            pltpu.VMEM_SHARED((_NP, _D), jnp.float32),
            pltpu.SemaphoreType.DMA,
            pltpu.SemaphoreType.DMA,
            pltpu.SemaphoreType.DMA,
            pltpu.SemaphoreType.DMA,
            pltpu.SemaphoreType.DMA,
            pltpu.SemaphoreType.DMA,
        ],
    )


_BN = 1000


def _norm_block(aggp_ref, degp_ref, x_ref, Wl_ref, bl_ref, Wr_ref, g_ref,
                b_ref):
    agg = aggp_ref[0] + aggp_ref[1]
    deg = degp_ref[0, :, pl.ds(0, 1)] + degp_ref[1, :, pl.ds(0, 1)]
    agg = agg / jnp.maximum(deg, 1.0)
    y = (lax.dot_general(agg, Wl_ref[...], (((1,), (1,)), ((), ())),
                         preferred_element_type=jnp.float32)
         + bl_ref[...]
         + lax.dot_general(x_ref[...], Wr_ref[...], (((1,), (1,)), ((), ())),
                           preferred_element_type=jnp.float32))
    mu = jnp.mean(y, axis=-1, keepdims=True)
    var = jnp.mean((y - mu) ** 2, axis=-1, keepdims=True)
    hn = g_ref[...] * (y - mu) / jnp.sqrt(var + 1e-5) + b_ref[...]
    return jnp.maximum(hn, 0.0)


def _dense_body(aggp_ref, degp_ref, x_ref, Wl_ref, bl_ref, Wr_ref, g_ref,
                b_ref, o_ref):
    o_ref[...] = _norm_block(aggp_ref, degp_ref, x_ref, Wl_ref, bl_ref,
                             Wr_ref, g_ref, b_ref)


def _final_body(aggp_ref, degp_ref, x_ref, Wl_ref, bl_ref, Wr_ref, g_ref,
                b_ref, W1_ref, b1_ref, W2_ref, b2_ref, W3_ref, b3_ref, o_ref):
    h = _norm_block(aggp_ref, degp_ref, x_ref, Wl_ref, bl_ref, Wr_ref, g_ref,
                    b_ref)
    t = jnp.maximum(
        lax.dot_general(h, W1_ref[...], (((1,), (1,)), ((), ())),
                        preferred_element_type=jnp.float32) + b1_ref[...], 0.0)
    t = jnp.maximum(
        lax.dot_general(t, W2_ref[...], (((1,), (1,)), ((), ())),
                        preferred_element_type=jnp.float32) + b2_ref[...], 0.0)
    o_ref[...] = (lax.dot_general(t, W3_ref[...], (((1,), (1,)), ((), ())),
                                  preferred_element_type=jnp.float32)
                  + b3_ref[...])


def _row_spec(k):
    return pl.BlockSpec((_BN, k), lambda i: (i, 0))


def _full_spec(shape):
    nd = len(shape)
    return pl.BlockSpec(shape, lambda i, _n=nd: (0,) * _n)


def _dense(aggp, degp, x, Wl, bl, Wr, g, b):
    return pl.pallas_call(
        _dense_body,
        grid=(_N // _BN,),
        in_specs=[
            pl.BlockSpec((_NC, _BN, _D), lambda i: (0, i, 0)),
            pl.BlockSpec((_NC, _BN, _D), lambda i: (0, i, 0)),
            _row_spec(_D),
            _full_spec((_H, _D)), _full_spec((1, _H)),
            _full_spec((_H, _D)), _full_spec((1, _H)), _full_spec((1, _H)),
        ],
        out_specs=_row_spec(_H),
        out_shape=jax.ShapeDtypeStruct((_N, _H), jnp.float32),
    )(aggp, degp, x, Wl, bl.reshape(1, _H), Wr, g.reshape(1, _H),
      b.reshape(1, _H))


def _dense_final(aggp, degp, x, Wl, bl, Wr, g, b, W1, b1, W2, b2, W3, b3):
    h2, h4, ol = _H // 2, _H // 4, 8
    return pl.pallas_call(
        _final_body,
        grid=(_N // _BN,),
        in_specs=[
            pl.BlockSpec((_NC, _BN, _D), lambda i: (0, i, 0)),
            pl.BlockSpec((_NC, _BN, _D), lambda i: (0, i, 0)),
            _row_spec(_D),
            _full_spec((_H, _D)), _full_spec((1, _H)),
            _full_spec((_H, _D)), _full_spec((1, _H)), _full_spec((1, _H)),
            _full_spec((h2, _H)), _full_spec((1, h2)),
            _full_spec((h4, h2)), _full_spec((1, h4)),
            _full_spec((ol, h4)), _full_spec((1, ol)),
        ],
        out_specs=_row_spec(ol),
        out_shape=jax.ShapeDtypeStruct((_N, ol), jnp.float32),
    )(aggp, degp, x, Wl, bl.reshape(1, _H), Wr, g.reshape(1, _H),
      b.reshape(1, _H), W1, b1.reshape(1, h2), W2, b2.reshape(1, h4),
      W3, b3.reshape(1, ol))


def kernel(x, edge_index, conv0_Wl, conv0_bl, conv0_Wr, norm0_g, norm0_b,
           conv1_Wl, conv1_bl, conv1_Wr, norm1_g, norm1_b,
           conv2_Wl, conv2_bl, conv2_Wr, norm2_g, norm2_b,
           reg_W1, reg_b1, reg_W2, reg_b2, reg_W3, reg_b3):
    src = edge_index[0]
    dst = edge_index[1]
    degp = _make_sc_deg()(dst)
    aggp = _make_sc_agg()(x, src, dst)
    h = _dense(aggp, degp, x, conv0_Wl, conv0_bl, conv0_Wr, norm0_g, norm0_b)
    aggp = _make_sc_agg()(h, src, dst)
    h = _dense(aggp, degp, h, conv1_Wl, conv1_bl, conv1_Wr, norm1_g, norm1_b)
    aggp = _make_sc_agg()(h, src, dst)
    return _dense_final(aggp, degp, h, conv2_Wl, conv2_bl, conv2_Wr, norm2_g,
                        norm2_b, reg_W1, reg_b1, reg_W2, reg_b2, reg_W3,
                        reg_b3)


# speedup vs baseline: 1.1215x; 1.1215x over previous
import functools

import jax
import jax.numpy as jnp
from jax import lax
from jax.experimental import pallas as pl
from jax.experimental.pallas import tpu as pltpu
from jax.experimental.pallas import tpu_sc as plsc

_N = 10000
_E = 320000
_D = 128
_H = 128

_NC = 2
_NS = 16
_L = 16
_NW = _NC * _NS
_EPW = _E // _NW
_C = 80
_NCHUNK = _EPW // _C
_NP = 10240
_U = 80
_RPT = _NP // _NS


def _sc_agg_body(h_hbm, src_hbm, dst_hbm, agg_out,
                 src_v0, dst_v0, rows_v0, src_v1, dst_v1, rows_v1,
                 agg_sh, sem_i0, sem_i1, sem_g0, sem_g1):
    zbuf = rows_v0

    cid = lax.axis_index("c")
    sid = lax.axis_index("s")
    wid = sid * _NC + cid

    zeros16 = jnp.zeros((_L,), jnp.float32)

    def _zb(i, c):
        r = i // (_D // _L)
        k = i % (_D // _L)
        zbuf[r, pl.ds(k * _L, _L)] = zeros16
        return c

    lax.fori_loop(0, _U * (_D // _L), _zb, 0)

    r0 = sid * _RPT
    for j in range(_RPT // _U):
        pltpu.sync_copy(zbuf, agg_sh.at[pl.ds(r0 + j * _U, _U), :])

    plsc.subcore_barrier()

    ebase = wid * _EPW
    emax = _E - _C

    bufs = ((src_v0, dst_v0, rows_v0, sem_i0, sem_g0),
            (src_v1, dst_v1, rows_v1, sem_i1, sem_g1))

    def _idx_start(j, b):
        src_v, dst_v, _, sem_i, _ = b
        off = jnp.minimum(ebase + j * _C, emax)
        ci0 = pltpu.async_copy(src_hbm.at[pl.ds(off, _C)], src_v, sem_i)
        ci1 = pltpu.async_copy(dst_hbm.at[pl.ds(off, _C)], dst_v, sem_i)
        return ci0, ci1

    def _idx_wait(cis):
        cis[0].wait()
        cis[1].wait()

    def _gather_start(b):
        src_v, _, rows_v, _, sem_g = b
        return pltpu.async_copy(h_hbm.at[src_v], rows_v, sem_g)

    def _scatter(b):
        _, dst_v, rows_v, _, _ = b
        pltpu.sync_copy(rows_v, agg_sh.at[dst_v], add=True)

    _idx_wait(_idx_start(0, bufs[0]))
    g0 = _gather_start(bufs[0])
    i1 = _idx_start(1, bufs[1])

    def _pair(i, c):
        a = 2 * i + 1
        _idx_wait(i1)
        g1 = _gather_start(bufs[1])
        g0.wait()
        _scatter(bufs[0])
        i0 = _idx_start(a + 1, bufs[0])
        _idx_wait(i0)
        g0b = _gather_start(bufs[0])
        g1.wait()
        _scatter(bufs[1])
        i1b = _idx_start(a + 2, bufs[1])
        return c

    lax.fori_loop(0, (_NCHUNK - 1) // 2, _pair, 0)

    _idx_wait(i1)
    g0.wait()
    _scatter(bufs[0])

    plsc.subcore_barrier()

    pltpu.sync_copy(agg_sh.at[pl.ds(r0, _RPT), :],
                    agg_out.at[cid, pl.ds(r0, _RPT), :])


def _sc_deg_body(dst_hbm, deg_out, dst_v0, dst_v1, ones_v, zbuf, deg_sh,
                 sem_i0, sem_i1):
    cid = lax.axis_index("c")
    sid = lax.axis_index("s")
    wid = sid * _NC + cid

    zeros16 = jnp.zeros((_L,), jnp.float32)
    ones16 = jnp.ones((_L,), jnp.float32)

    def _fb(i, c):
        r = i // (_D // _L)
        k = i % (_D // _L)
        zbuf[r, pl.ds(k * _L, _L)] = zeros16
        ones_v[r, pl.ds(k * _L, _L)] = ones16
        return c

    lax.fori_loop(0, _U * (_D // _L), _fb, 0)

    r0 = sid * _RPT
    for j in range(_RPT // _U):
        pltpu.sync_copy(zbuf, deg_sh.at[pl.ds(r0 + j * _U, _U), :])

    plsc.subcore_barrier()

    ebase = wid * _EPW
    emax = _E - _C

    def _idx_start(j, dst_v, sem_i):
        off = jnp.minimum(ebase + j * _C, emax)
        return pltpu.async_copy(dst_hbm.at[pl.ds(off, _C)], dst_v, sem_i)

    i0 = _idx_start(0, dst_v0, sem_i0)
    i1 = _idx_start(1, dst_v1, sem_i1)

    def _pair(i, c):
        a = 2 * i
        i0.wait()
        pltpu.sync_copy(ones_v, deg_sh.at[dst_v0], add=True)
        _idx_start(a + 2, dst_v0, sem_i0)
        i1.wait()
        pltpu.sync_copy(ones_v, deg_sh.at[dst_v1], add=True)
        _idx_start(a + 3, dst_v1, sem_i1)
        return c

    lax.fori_loop(0, (_NCHUNK - 1) // 2, _pair, 0)

    i0.wait()
    pltpu.sync_copy(ones_v, deg_sh.at[dst_v0], add=True)
    i1.wait()

    plsc.subcore_barrier()

    pltpu.sync_copy(deg_sh.at[pl.ds(r0, _RPT), :],
                    deg_out.at[cid, pl.ds(r0, _RPT), :])


@functools.lru_cache(maxsize=None)
def _make_sc_deg():
    mesh = plsc.VectorSubcoreMesh(core_axis_name="c", subcore_axis_name="s",
                                  num_cores=_NC, num_subcores=_NS)
    return pl.kernel(
        _sc_deg_body,
        out_type=jax.ShapeDtypeStruct((_NC, _NP, _D), jnp.float32),
        mesh=mesh,
        scratch_types=[
            pltpu.VMEM((_C,), jnp.int32),
            pltpu.VMEM((_C,), jnp.int32),
            pltpu.VMEM((_C, _D), jnp.float32),
            pltpu.VMEM((_U, _D), jnp.float32),
            pltpu.VMEM_SHARED((_NP, _D), jnp.float32),
            pltpu.SemaphoreType.DMA,
            pltpu.SemaphoreType.DMA,
        ],
    )


@functools.lru_cache(maxsize=None)
def _make_sc_agg():
    mesh = plsc.VectorSubcoreMesh(core_axis_name="c", subcore_axis_name="s",
                                  num_cores=_NC, num_subcores=_NS)
    return pl.kernel(
        _sc_agg_body,
        out_type=jax.ShapeDtypeStruct((_NC, _NP, _D), jnp.float32),
        mesh=mesh,
        scratch_types=[
            pltpu.VMEM((_C,), jnp.int32),
            pltpu.VMEM((_C,), jnp.int32),
            pltpu.VMEM((_C, _D), jnp.float32),
            pltpu.VMEM((_C,), jnp.int32),
            pltpu.VMEM((_C,), jnp.int32),
            pltpu.VMEM((_C, _D), jnp.float32),
            pltpu.VMEM_SHARED((_NP, _D), jnp.float32),
            pltpu.SemaphoreType.DMA,
            pltpu.SemaphoreType.DMA,
            pltpu.SemaphoreType.DMA,
            pltpu.SemaphoreType.DMA,
        ],
    )


_BN = 1000


def _norm_block(aggp_ref, degp_ref, x_ref, Wl_ref, bl_ref, Wr_ref, g_ref,
                b_ref):
    agg = aggp_ref[0] + aggp_ref[1]
    deg = degp_ref[0, :, pl.ds(0, 1)] + degp_ref[1, :, pl.ds(0, 1)]
    agg = agg / jnp.maximum(deg, 1.0)
    y = (lax.dot_general(agg, Wl_ref[...], (((1,), (1,)), ((), ())),
                         preferred_element_type=jnp.float32)
         + bl_ref[...]
         + lax.dot_general(x_ref[...], Wr_ref[...], (((1,), (1,)), ((), ())),
                           preferred_element_type=jnp.float32))
    mu = jnp.mean(y, axis=-1, keepdims=True)
    var = jnp.mean((y - mu) ** 2, axis=-1, keepdims=True)
    hn = g_ref[...] * (y - mu) / jnp.sqrt(var + 1e-5) + b_ref[...]
    return jnp.maximum(hn, 0.0)


def _dense_body(aggp_ref, degp_ref, x_ref, Wl_ref, bl_ref, Wr_ref, g_ref,
                b_ref, o_ref):
    o_ref[...] = _norm_block(aggp_ref, degp_ref, x_ref, Wl_ref, bl_ref,
                             Wr_ref, g_ref, b_ref)


def _final_body(aggp_ref, degp_ref, x_ref, Wl_ref, bl_ref, Wr_ref, g_ref,
                b_ref, W1_ref, b1_ref, W2_ref, b2_ref, W3_ref, b3_ref, o_ref):
    h = _norm_block(aggp_ref, degp_ref, x_ref, Wl_ref, bl_ref, Wr_ref, g_ref,
                    b_ref)
    t = jnp.maximum(
        lax.dot_general(h, W1_ref[...], (((1,), (1,)), ((), ())),
                        preferred_element_type=jnp.float32) + b1_ref[...], 0.0)
    t = jnp.maximum(
        lax.dot_general(t, W2_ref[...], (((1,), (1,)), ((), ())),
                        preferred_element_type=jnp.float32) + b2_ref[...], 0.0)
    o_ref[...] = (lax.dot_general(t, W3_ref[...], (((1,), (1,)), ((), ())),
                                  preferred_element_type=jnp.float32)
                  + b3_ref[...])


def _row_spec(k):
    return pl.BlockSpec((_BN, k), lambda i: (i, 0))


def _full_spec(shape):
    nd = len(shape)
    return pl.BlockSpec(shape, lambda i, _n=nd: (0,) * _n)


def _dense(aggp, degp, x, Wl, bl, Wr, g, b):
    return pl.pallas_call(
        _dense_body,
        grid=(_N // _BN,),
        in_specs=[
            pl.BlockSpec((_NC, _BN, _D), lambda i: (0, i, 0)),
            pl.BlockSpec((_NC, _BN, _D), lambda i: (0, i, 0)),
            _row_spec(_D),
            _full_spec((_H, _D)), _full_spec((1, _H)),
            _full_spec((_H, _D)), _full_spec((1, _H)), _full_spec((1, _H)),
        ],
        out_specs=_row_spec(_H),
        out_shape=jax.ShapeDtypeStruct((_N, _H), jnp.float32),
    )(aggp, degp, x, Wl, bl.reshape(1, _H), Wr, g.reshape(1, _H),
      b.reshape(1, _H))


def _dense_final(aggp, degp, x, Wl, bl, Wr, g, b, W1, b1, W2, b2, W3, b3):
    h2, h4, ol = _H // 2, _H // 4, 8
    return pl.pallas_call(
        _final_body,
        grid=(_N // _BN,),
        in_specs=[
            pl.BlockSpec((_NC, _BN, _D), lambda i: (0, i, 0)),
            pl.BlockSpec((_NC, _BN, _D), lambda i: (0, i, 0)),
            _row_spec(_D),
            _full_spec((_H, _D)), _full_spec((1, _H)),
            _full_spec((_H, _D)), _full_spec((1, _H)), _full_spec((1, _H)),
            _full_spec((h2, _H)), _full_spec((1, h2)),
            _full_spec((h4, h2)), _full_spec((1, h4)),
            _full_spec((ol, h4)), _full_spec((1, ol)),
        ],
        out_specs=_row_spec(ol),
        out_shape=jax.ShapeDtypeStruct((_N, ol), jnp.float32),
    )(aggp, degp, x, Wl, bl.reshape(1, _H), Wr, g.reshape(1, _H),
      b.reshape(1, _H), W1, b1.reshape(1, h2), W2, b2.reshape(1, h4),
      W3, b3.reshape(1, ol))


def kernel(x, edge_index, conv0_Wl, conv0_bl, conv0_Wr, norm0_g, norm0_b,
           conv1_Wl, conv1_bl, conv1_Wr, norm1_g, norm1_b,
           conv2_Wl, conv2_bl, conv2_Wr, norm2_g, norm2_b,
           reg_W1, reg_b1, reg_W2, reg_b2, reg_W3, reg_b3):
    src = edge_index[0]
    dst = edge_index[1]
    degp = _make_sc_deg()(dst)
    aggp = _make_sc_agg()(x, src, dst)
    h = _dense(aggp, degp, x, conv0_Wl, conv0_bl, conv0_Wr, norm0_g, norm0_b)
    aggp = _make_sc_agg()(h, src, dst)
    h = _dense(aggp, degp, h, conv1_Wl, conv1_bl, conv1_Wr, norm1_g, norm1_b)
    aggp = _make_sc_agg()(h, src, dst)
    return _dense_final(aggp, degp, h, conv2_Wl, conv2_bl, conv2_Wr, norm2_g,
                        norm2_b, reg_W1, reg_b1, reg_W2, reg_b2, reg_W3,
                        reg_b3)


# speedup vs baseline: 1.2579x; 1.1217x over previous
import functools

import jax
import jax.numpy as jnp
from jax import lax
from jax.experimental import pallas as pl
from jax.experimental.pallas import tpu as pltpu
from jax.experimental.pallas import tpu_sc as plsc

_N = 10000
_E = 320000
_D = 128
_H = 128

_NC = 2
_NS = 16
_L = 16
_NW = _NC * _NS
_EPW = _E // _NW
_C = 80
_NCHUNK = _EPW // _C
_CA = 128
_NCA = _EPW // _CA
_TAIL = _EPW - _NCA * _CA
_NP = 10240
_U = 80
_RPT = _NP // _NS


def _sc_agg_body(h_hbm, src_hbm, dst_hbm, agg_out,
                 src_v0, dst_v0, rows_v0, src_v1, dst_v1, rows_v1,
                 tsrc_v, tdst_v, trows_v,
                 agg_sh, sem_i0, sem_i1, sem_g0, sem_g1):
    zbuf = rows_v0

    cid = lax.axis_index("c")
    sid = lax.axis_index("s")
    wid = sid * _NC + cid

    zeros16 = jnp.zeros((_L,), jnp.float32)

    def _zb(i, c):
        r = i // (_D // _L)
        k = i % (_D // _L)
        zbuf[r, pl.ds(k * _L, _L)] = zeros16
        return c

    lax.fori_loop(0, _CA * (_D // _L), _zb, 0)

    r0 = sid * _RPT
    for j in range(_RPT // _CA):
        pltpu.sync_copy(zbuf, agg_sh.at[pl.ds(r0 + j * _CA, _CA), :])

    plsc.subcore_barrier()

    ebase = wid * _EPW

    bufs = ((src_v0, dst_v0, rows_v0, sem_i0, sem_g0),
            (src_v1, dst_v1, rows_v1, sem_i1, sem_g1))

    def _idx_start(j, b):
        off = ebase + j * _CA
        ci0 = pltpu.async_copy(src_hbm.at[pl.ds(off, _CA)], b[0], b[3])
        ci1 = pltpu.async_copy(dst_hbm.at[pl.ds(off, _CA)], b[1], b[3])
        return ci0, ci1

    def _idx_wait(cis):
        cis[0].wait()
        cis[1].wait()

    def _gather_start(b):
        return pltpu.async_copy(h_hbm.at[b[0]], b[2], b[4])

    def _scatter(b):
        pltpu.sync_copy(b[2], agg_sh.at[b[1]], add=True)

    _idx_wait(_idx_start(0, bufs[0]))
    g0 = _gather_start(bufs[0])
    i1 = _idx_start(1, bufs[1])

    def _pair(i, c):
        a = 2 * i + 1
        _idx_wait(i1)
        g1 = _gather_start(bufs[1])
        g0.wait()
        _scatter(bufs[0])
        i0 = _idx_start(a + 1, bufs[0])
        _idx_wait(i0)
        g0b = _gather_start(bufs[0])
        g1.wait()
        _scatter(bufs[1])
        i1b = _idx_start(a + 2, bufs[1])
        return c

    lax.fori_loop(0, (_NCA - 2) // 2, _pair, 0)

    _idx_wait(i1)
    g1 = _gather_start(bufs[1])
    g0.wait()
    _scatter(bufs[0])
    g1.wait()
    _scatter(bufs[1])
    toff = ebase + _NCA * _CA
    pltpu.sync_copy(src_hbm.at[pl.ds(toff, _TAIL)], tsrc_v)
    pltpu.sync_copy(dst_hbm.at[pl.ds(toff, _TAIL)], tdst_v)
    pltpu.async_copy(h_hbm.at[tsrc_v], trows_v, sem_g0).wait()
    pltpu.sync_copy(trows_v, agg_sh.at[tdst_v], add=True)

    plsc.subcore_barrier()

    pltpu.sync_copy(agg_sh.at[pl.ds(r0, _RPT), :],
                    agg_out.at[cid, pl.ds(r0, _RPT), :])


def _sc_deg_body(dst_hbm, deg_out, dst_v0, dst_v1, ones_v, zbuf, deg_sh,
                 sem_i0, sem_i1):
    cid = lax.axis_index("c")
    sid = lax.axis_index("s")
    wid = sid * _NC + cid

    zeros16 = jnp.zeros((_L,), jnp.float32)
    ones16 = jnp.ones((_L,), jnp.float32)

    def _fb(i, c):
        r = i // (_D // _L)
        k = i % (_D // _L)
        zbuf[r, pl.ds(k * _L, _L)] = zeros16
        ones_v[r, pl.ds(k * _L, _L)] = ones16
        return c

    lax.fori_loop(0, _U * (_D // _L), _fb, 0)

    r0 = sid * _RPT
    for j in range(_RPT // _U):
        pltpu.sync_copy(zbuf, deg_sh.at[pl.ds(r0 + j * _U, _U), :])

    plsc.subcore_barrier()

    ebase = wid * _EPW
    emax = _E - _C

    def _idx_start(j, dst_v, sem_i):
        off = jnp.minimum(ebase + j * _C, emax)
        return pltpu.async_copy(dst_hbm.at[pl.ds(off, _C)], dst_v, sem_i)

    i0 = _idx_start(0, dst_v0, sem_i0)
    i1 = _idx_start(1, dst_v1, sem_i1)

    def _pair(i, c):
        a = 2 * i
        i0.wait()
        pltpu.sync_copy(ones_v, deg_sh.at[dst_v0], add=True)
        _idx_start(a + 2, dst_v0, sem_i0)
        i1.wait()
        pltpu.sync_copy(ones_v, deg_sh.at[dst_v1], add=True)
        _idx_start(a + 3, dst_v1, sem_i1)
        return c

    lax.fori_loop(0, (_NCHUNK - 1) // 2, _pair, 0)

    i0.wait()
    pltpu.sync_copy(ones_v, deg_sh.at[dst_v0], add=True)
    i1.wait()

    plsc.subcore_barrier()

    pltpu.sync_copy(deg_sh.at[pl.ds(r0, _RPT), :],
                    deg_out.at[cid, pl.ds(r0, _RPT), :])


@functools.lru_cache(maxsize=None)
def _make_sc_deg():
    mesh = plsc.VectorSubcoreMesh(core_axis_name="c", subcore_axis_name="s",
                                  num_cores=_NC, num_subcores=_NS)
    return pl.kernel(
        _sc_deg_body,
        out_type=jax.ShapeDtypeStruct((_NC, _NP, _D), jnp.float32),
        mesh=mesh,
        scratch_types=[
            pltpu.VMEM((_C,), jnp.int32),
            pltpu.VMEM((_C,), jnp.int32),
            pltpu.VMEM((_C, _D), jnp.float32),
            pltpu.VMEM((_U, _D), jnp.float32),
            pltpu.VMEM_SHARED((_NP, _D), jnp.float32),
            pltpu.SemaphoreType.DMA,
            pltpu.SemaphoreType.DMA,
        ],
    )


@functools.lru_cache(maxsize=None)
def _make_sc_agg():
    mesh = plsc.VectorSubcoreMesh(core_axis_name="c", subcore_axis_name="s",
                                  num_cores=_NC, num_subcores=_NS)
    return pl.kernel(
        _sc_agg_body,
        out_type=jax.ShapeDtypeStruct((_NC, _NP, _D), jnp.float32),
        mesh=mesh,
        scratch_types=[
            pltpu.VMEM((_CA,), jnp.int32),
            pltpu.VMEM((_CA,), jnp.int32),
            pltpu.VMEM((_CA, _D), jnp.float32),
            pltpu.VMEM((_CA,), jnp.int32),
            pltpu.VMEM((_CA,), jnp.int32),
            pltpu.VMEM((_CA, _D), jnp.float32),
            pltpu.VMEM((_TAIL,), jnp.int32),
            pltpu.VMEM((_TAIL,), jnp.int32),
            pltpu.VMEM((_TAIL, _D), jnp.float32),
            pltpu.VMEM_SHARED((_NP, _D), jnp.float32),
            pltpu.SemaphoreType.DMA,
            pltpu.SemaphoreType.DMA,
            pltpu.SemaphoreType.DMA,
            pltpu.SemaphoreType.DMA,
        ],
    )


_BN = 1000


def _norm_block(aggp_ref, degp_ref, x_ref, Wl_ref, bl_ref, Wr_ref, g_ref,
                b_ref):
    agg = aggp_ref[0] + aggp_ref[1]
    deg = degp_ref[0, :, pl.ds(0, 1)] + degp_ref[1, :, pl.ds(0, 1)]
    agg = agg / jnp.maximum(deg, 1.0)
    y = (lax.dot_general(agg, Wl_ref[...], (((1,), (1,)), ((), ())),
                         preferred_element_type=jnp.float32)
         + bl_ref[...]
         + lax.dot_general(x_ref[...], Wr_ref[...], (((1,), (1,)), ((), ())),
                           preferred_element_type=jnp.float32))
    mu = jnp.mean(y, axis=-1, keepdims=True)
    var = jnp.mean((y - mu) ** 2, axis=-1, keepdims=True)
    hn = g_ref[...] * (y - mu) / jnp.sqrt(var + 1e-5) + b_ref[...]
    return jnp.maximum(hn, 0.0)


def _dense_body(aggp_ref, degp_ref, x_ref, Wl_ref, bl_ref, Wr_ref, g_ref,
                b_ref, o_ref):
    o_ref[...] = _norm_block(aggp_ref, degp_ref, x_ref, Wl_ref, bl_ref,
                             Wr_ref, g_ref, b_ref)


def _final_body(aggp_ref, degp_ref, x_ref, Wl_ref, bl_ref, Wr_ref, g_ref,
                b_ref, W1_ref, b1_ref, W2_ref, b2_ref, W3_ref, b3_ref, o_ref):
    h = _norm_block(aggp_ref, degp_ref, x_ref, Wl_ref, bl_ref, Wr_ref, g_ref,
                    b_ref)
    t = jnp.maximum(
        lax.dot_general(h, W1_ref[...], (((1,), (1,)), ((), ())),
                        preferred_element_type=jnp.float32) + b1_ref[...], 0.0)
    t = jnp.maximum(
        lax.dot_general(t, W2_ref[...], (((1,), (1,)), ((), ())),
                        preferred_element_type=jnp.float32) + b2_ref[...], 0.0)
    o_ref[...] = (lax.dot_general(t, W3_ref[...], (((1,), (1,)), ((), ())),
                                  preferred_element_type=jnp.float32)
                  + b3_ref[...])


def _row_spec(k):
    return pl.BlockSpec((_BN, k), lambda i: (i, 0))


def _full_spec(shape):
    nd = len(shape)
    return pl.BlockSpec(shape, lambda i, _n=nd: (0,) * _n)


def _dense(aggp, degp, x, Wl, bl, Wr, g, b):
    return pl.pallas_call(
        _dense_body,
        grid=(_N // _BN,),
        in_specs=[
            pl.BlockSpec((_NC, _BN, _D), lambda i: (0, i, 0)),
            pl.BlockSpec((_NC, _BN, _D), lambda i: (0, i, 0)),
            _row_spec(_D),
            _full_spec((_H, _D)), _full_spec((1, _H)),
            _full_spec((_H, _D)), _full_spec((1, _H)), _full_spec((1, _H)),
        ],
        out_specs=_row_spec(_H),
        out_shape=jax.ShapeDtypeStruct((_N, _H), jnp.float32),
    )(aggp, degp, x, Wl, bl.reshape(1, _H), Wr, g.reshape(1, _H),
      b.reshape(1, _H))


def _dense_final(aggp, degp, x, Wl, bl, Wr, g, b, W1, b1, W2, b2, W3, b3):
    h2, h4, ol = _H // 2, _H // 4, 8
    return pl.pallas_call(
        _final_body,
        grid=(_N // _BN,),
        in_specs=[
            pl.BlockSpec((_NC, _BN, _D), lambda i: (0, i, 0)),
            pl.BlockSpec((_NC, _BN, _D), lambda i: (0, i, 0)),
            _row_spec(_D),
            _full_spec((_H, _D)), _full_spec((1, _H)),
            _full_spec((_H, _D)), _full_spec((1, _H)), _full_spec((1, _H)),
            _full_spec((h2, _H)), _full_spec((1, h2)),
            _full_spec((h4, h2)), _full_spec((1, h4)),
            _full_spec((ol, h4)), _full_spec((1, ol)),
        ],
        out_specs=_row_spec(ol),
        out_shape=jax.ShapeDtypeStruct((_N, ol), jnp.float32),
    )(aggp, degp, x, Wl, bl.reshape(1, _H), Wr, g.reshape(1, _H),
      b.reshape(1, _H), W1, b1.reshape(1, h2), W2, b2.reshape(1, h4),
      W3, b3.reshape(1, ol))


def kernel(x, edge_index, conv0_Wl, conv0_bl, conv0_Wr, norm0_g, norm0_b,
           conv1_Wl, conv1_bl, conv1_Wr, norm1_g, norm1_b,
           conv2_Wl, conv2_bl, conv2_Wr, norm2_g, norm2_b,
           reg_W1, reg_b1, reg_W2, reg_b2, reg_W3, reg_b3):
    src = edge_index[0]
    dst = edge_index[1]
    degp = _make_sc_deg()(dst)
    aggp = _make_sc_agg()(x, src, dst)
    h = _dense(aggp, degp, x, conv0_Wl, conv0_bl, conv0_Wr, norm0_g, norm0_b)
    aggp = _make_sc_agg()(h, src, dst)
    h = _dense(aggp, degp, h, conv1_Wl, conv1_bl, conv1_Wr, norm1_g, norm1_b)
    aggp = _make_sc_agg()(h, src, dst)
    return _dense_final(aggp, degp, h, conv2_Wl, conv2_bl, conv2_Wr, norm2_g,
                        norm2_b, reg_W1, reg_b1, reg_W2, reg_b2, reg_W3,
                        reg_b3)
